# Initial kernel scaffold; baseline (speedup 1.0000x reference)
import jax
import jax.numpy as jnp
from jax.experimental import pallas as pl


def kernel(x):
    raise NotImplementedError("write your pallas kernel here")


# speedup vs baseline: 108.8773x; 108.8773x over previous
import dataclasses
import functools

import jax
import jax.numpy as jnp
from jax import lax
from jax.experimental import pallas as pl
from jax.experimental.pallas import tpu as pltpu
from jax.experimental.pallas import tpu_sc as plsc

N_BINS = 16
CHUNK = 2048


def _encode_block(x_vmem, o_vmem):
    row16 = lax.iota(jnp.int32, 16) * 16

    @pl.loop(0, CHUNK, step=16)
    def _(g):
        xv = x_vmem[pl.ds(g, 16)]
        u = xv * (8.0 / 3.0) + 8.0
        base = row16 + g * 16
        for r in range(N_BINS):
            v = u - float(r)
            if r < N_BINS - 1:
                v = jnp.minimum(v, 1.0)
            if r > 0:
                v = jnp.maximum(v, 0.0)
            plsc.store_scatter(o_vmem, [base + r], v)


def kernel(x):
    b, f = x.shape
    n = b * f
    xf = x.reshape(n)
    mesh = plsc.VectorSubcoreMesh(core_axis_name="c", subcore_axis_name="s")
    cp = pltpu.CompilerParams()
    if "needs_layout_passes" in pltpu.CompilerParams.__dataclass_fields__:
        cp = dataclasses.replace(cp, needs_layout_passes=False)

    @functools.partial(
        pl.kernel,
        out_type=jax.ShapeDtypeStruct((n * N_BINS,), jnp.float32),
        mesh=mesh,
        compiler_params=cp,
    )
    def run(x_hbm, o_hbm):
        pltpu.emit_pipeline(
            _encode_block,
            grid=(n // CHUNK,),
            in_specs=[pl.BlockSpec((CHUNK,), lambda i: (i,))],
            out_specs=[pl.BlockSpec((CHUNK * N_BINS,), lambda i: (i,))],
            core_axis_name=("c", "s"),
            dimension_semantics=(pltpu.PARALLEL,),
        )(x_hbm, o_hbm)

    return run(xf).reshape(b, f, N_BINS)


# speedup vs baseline: 1746.5640x; 16.0416x over previous
import functools

import jax
import jax.numpy as jnp
from jax.experimental import pallas as pl
from jax.experimental.pallas import tpu as pltpu
from jax.experimental.pallas import tpu_sc as plsc

N_BINS = 16
BBLK = 2048


def _encode_block(x_vmem, o_vmem):
    @pl.loop(0, BBLK, step=16)
    def _(c):
        xv = x_vmem[0, pl.ds(c, 16)]
        u = xv * (8.0 / 3.0) + 8.0
        for r in range(N_BINS):
            v = u - float(r)
            if r < N_BINS - 1:
                v = jnp.minimum(v, 1.0)
            if r > 0:
                v = jnp.maximum(v, 0.0)
            o_vmem[r, pl.ds(c, 16)] = v


def kernel(x):
    b, f = x.shape
    xt = x.T
    mesh = plsc.VectorSubcoreMesh(core_axis_name="c", subcore_axis_name="s")

    @functools.partial(
        pl.kernel,
        out_type=jax.ShapeDtypeStruct((f * N_BINS, b), jnp.float32),
        mesh=mesh,
    )
    def run(x_hbm, o_hbm):
        pltpu.emit_pipeline(
            _encode_block,
            grid=(f, b // BBLK),
            in_specs=[pl.BlockSpec((1, BBLK), lambda i, j: (i, j))],
            out_specs=[pl.BlockSpec((N_BINS, BBLK), lambda i, j: (i, j))],
            core_axis_name=("c", "s"),
            dimension_semantics=(pltpu.PARALLEL, pltpu.PARALLEL),
        )(x_hbm, o_hbm)

    out = run(xt)
    return out.reshape(f, N_BINS, b).transpose(2, 0, 1)


# speedup vs baseline: 2158.3452x; 1.2358x over previous
import functools

import jax
import jax.numpy as jnp
from jax.experimental import pallas as pl
from jax.experimental.pallas import tpu as pltpu
from jax.experimental.pallas import tpu_sc as plsc

N_BINS = 16
BBLK = 2048


def _encode_block(x_vmem, o_vmem):
    @plsc.parallel_loop(0, BBLK, 16, unroll=2)
    def _(c):
        xv = x_vmem[0, pl.ds(c, 16)]
        u = xv * (8.0 / 3.0) + 8.0
        for r in range(N_BINS):
            v = u - float(r)
            if r < N_BINS - 1:
                v = jnp.minimum(v, 1.0)
            if r > 0:
                v = jnp.maximum(v, 0.0)
            o_vmem[r, pl.ds(c, 16)] = v


def kernel(x):
    b, f = x.shape
    xt = x.T
    mesh = plsc.VectorSubcoreMesh(core_axis_name="c", subcore_axis_name="s")

    @functools.partial(
        pl.kernel,
        out_type=jax.ShapeDtypeStruct((f * N_BINS, b), jnp.float32),
        mesh=mesh,
    )
    def run(x_hbm, o_hbm):
        pltpu.emit_pipeline(
            _encode_block,
            grid=(f, b // BBLK),
            in_specs=[pl.BlockSpec((1, BBLK), lambda i, j: (i, j))],
            out_specs=[pl.BlockSpec((N_BINS, BBLK), lambda i, j: (i, j))],
            core_axis_name=("c", "s"),
            dimension_semantics=(pltpu.PARALLEL, pltpu.PARALLEL),
        )(x_hbm, o_hbm)

    out = run(xt)
    return out.reshape(f, N_BINS, b).transpose(2, 0, 1)
